# baseline (speedup 1.0000x reference)
import functools

import jax
import jax.numpy as jnp
from jax import lax
from jax.experimental import pallas as pl
from jax.experimental.pallas import tpu as pltpu
from jax.experimental.pallas import tpu_sc as plsc

B = 64
C_IN = 3
HW = 224
D = 768
PATCH = 16
GRID_HW = HW // PATCH
NPATCH = GRID_HW * GRID_HW
K_DB = 50000
REPORT_LEN = 100
TGT_LEN = 80
VOCAB = 2000

KB = 5000
NKB = K_DB // KB


def _feat_body(x_ref, w_ref, b_ref, o_ref):
    x = x_ref[0].astype(jnp.bfloat16)
    x6 = x.reshape(C_IN, GRID_HW, PATCH, GRID_HW, PATCH)
    p = x6.transpose(1, 3, 0, 2, 4).reshape(NPATCH, D)
    acc = lax.dot_general(p, w_ref[...], (((1,), (1,)), ((), ())),
                          preferred_element_type=jnp.float32)
    acc = acc + b_ref[...]
    o_ref[0] = jnp.max(acc, axis=0, keepdims=True)


def _features(images5, w2, bias2d):
    n = images5.shape[0]
    out = pl.pallas_call(
        _feat_body,
        grid=(n,),
        in_specs=[
            pl.BlockSpec((1, C_IN, GRID_HW, PATCH, HW), lambda b_: (b_, 0, 0, 0, 0)),
            pl.BlockSpec((D, D), lambda b_: (0, 0)),
            pl.BlockSpec((1, D), lambda b_: (0, 0)),
        ],
        out_specs=pl.BlockSpec((1, 1, D), lambda b_: (b_, 0, 0)),
        out_shape=jax.ShapeDtypeStruct((n, 1, D), jnp.float32),
    )(images5, w2, bias2d)
    return out.reshape(n, D)


def _dist_body(f_ref, a_ref, bm_ref, ba_ref):
    k = pl.program_id(0)
    a = a_ref[...]
    st = lax.dot_general(a, f_ref[...], (((1,), (1,)), ((), ())),
                         preferred_element_type=jnp.float32)
    s = jnp.sum(a * a, axis=1, keepdims=True) - 2.0 * st
    m = jnp.min(s, axis=0, keepdims=True)
    gidx = lax.broadcasted_iota(jnp.int32, s.shape, 0) + k * KB
    lidx = jnp.min(jnp.where(s <= m, gidx, jnp.int32(2 ** 30)),
                   axis=0, keepdims=True)
    bm_ref[0] = m
    ba_ref[0] = lidx


def _block_minima(feats, all_features):
    return pl.pallas_call(
        _dist_body,
        grid=(NKB,),
        in_specs=[
            pl.BlockSpec((B, D), lambda k: (0, 0)),
            pl.BlockSpec((KB, D), lambda k: (k, 0)),
        ],
        out_specs=[
            pl.BlockSpec((1, 1, B), lambda k: (k, 0, 0)),
            pl.BlockSpec((1, 1, B), lambda k: (k, 0, 0)),
        ],
        out_shape=[
            jax.ShapeDtypeStruct((NKB, 1, B), jnp.float32),
            jax.ShapeDtypeStruct((NKB, 1, B), jnp.int32),
        ],
    )(feats, all_features)


IMGS_W = 16
NW_ACT = B // IMGS_W


@functools.cache
def _sc_merge_kernel():
    mesh = plsc.VectorSubcoreMesh(core_axis_name="c", subcore_axis_name="s")

    @functools.partial(
        pl.kernel,
        mesh=mesh,
        out_type=jax.ShapeDtypeStruct((B,), jnp.int32),
        compiler_params=pltpu.CompilerParams(
            use_tc_tiling_on_sc=False, needs_layout_passes=False),
        scratch_types=[
            pltpu.VMEM((NKB, IMGS_W), jnp.float32),
            pltpu.VMEM((NKB, IMGS_W), jnp.int32),
            pltpu.VMEM((IMGS_W,), jnp.int32),
        ],
    )
    def merge(bm_hbm, ba_hbm, out_hbm, bm_v, ba_v, res_v):
        wid = lax.axis_index("s") * 2 + lax.axis_index("c")

        @pl.when(wid < NW_ACT)
        def _():
            base = wid * IMGS_W
            pltpu.sync_copy(bm_hbm.at[:, pl.ds(base, IMGS_W)], bm_v)
            pltpu.sync_copy(ba_hbm.at[:, pl.ds(base, IMGS_W)], ba_v)
            best = bm_v[0, :]
            bidx = ba_v[0, :]
            for nb in range(1, NKB):
                v = bm_v[nb, :]
                i = ba_v[nb, :]
                upd = v < best
                best = jnp.where(upd, v, best)
                bidx = jnp.where(upd, i, bidx)
            res_v[...] = bidx
            pltpu.sync_copy(res_v, out_hbm.at[pl.ds(base, IMGS_W)])

    return merge


def _sc_merge(bmins, bargs):
    return _sc_merge_kernel()(bmins.reshape(NKB, B), bargs.reshape(NKB, B))


OB = 8


def _onehot_body(cl_ref, *refs):
    ids_refs, o_ref = refs[:OB], refs[OB]
    b_ = pl.program_id(0)
    iot = lax.broadcasted_iota(jnp.int32, (TGT_LEN, VOCAB), 1)
    for i in range(OB):
        r8 = cl_ref[OB * b_ + i] % 8
        row = ids_refs[i][pl.ds(r8, 1), :TGT_LEN]
        o_ref[i] = (iot == jnp.transpose(row)).astype(jnp.float32)


def _onehot(closest, table):
    grid_spec = pltpu.PrefetchScalarGridSpec(
        num_scalar_prefetch=1,
        grid=(B // OB,),
        in_specs=[
            pl.BlockSpec((8, REPORT_LEN),
                         lambda b_, cl, i=i: (cl[OB * b_ + i] // 8, 0))
            for i in range(OB)
        ],
        out_specs=pl.BlockSpec((OB, TGT_LEN, VOCAB), lambda b_, cl: (b_, 0, 0)),
    )
    return pl.pallas_call(
        _onehot_body,
        grid_spec=grid_spec,
        out_shape=jax.ShapeDtypeStruct((B, TGT_LEN, VOCAB), jnp.float32),
    )(closest, *([table] * OB))


def kernel(images, W, b, all_features, all_reports, reports):
    del reports
    w2 = W.reshape(D, C_IN * PATCH * PATCH).astype(jnp.bfloat16)
    bias2d = b.reshape(1, D).astype(jnp.float32)

    images5 = images.reshape(B, C_IN, GRID_HW, PATCH, HW)
    feats = _features(images5, w2, bias2d)
    bmins, bargs = _block_minima(feats, all_features)
    closest = _sc_merge(bmins, bargs)
    table = all_reports.astype(jnp.int32)
    out = _onehot(closest, table)
    return (out,)
